# baseline (speedup 1.0000x reference)
import functools

import jax
import jax.numpy as jnp
from jax import lax
from jax.experimental import pallas as pl
from jax.experimental.pallas import tpu as pltpu
from jax.experimental.pallas import tpu_sc as plsc

_NC = 2
_NS = 16
_NW = _NC * _NS

_V = 1000000
_D = 64
_B = 4096
_H = 200

_VB = 128
_NFULL = _V // _VB
_TAIL = _V - _NFULL * _VB
_XTRA = _NFULL - (_NFULL // _NW) * _NW
_TASKS = (_H * (_B // 128)) // _NW

_CPARAMS = pltpu.CompilerParams(
    use_tc_tiling_on_sc=True, needs_layout_passes=False)


def _pair_transp(src, dst, chunks):
    iota = lax.iota(jnp.int32, 16)
    lcs = [iota + 16 * c for c in range(chunks)]
    prs = [(iota + 16 * c) // 2 for c in range(chunks)]
    colb = (iota % 2) * 64

    @plsc.parallel_loop(0, 16, unroll=4)
    def diag(d0):
        dd = (iota + d0) % 16
        for dh in range(_D // 16):
            dvec = dd + dh * 16
            colv = colb + dvec
            xs = [plsc.load_gather(src, [dvec, lcs[c]]) for c in range(chunks)]
            for c in range(chunks):
                plsc.store_scatter(dst, [prs[c], colv], xs[c])


@functools.cache
def _build_transpose():
    mesh = plsc.VectorSubcoreMesh(core_axis_name="c", subcore_axis_name="s")

    @functools.partial(
        pl.kernel,
        mesh=mesh,
        out_type=jax.ShapeDtypeStruct((_V // 2, 128), jnp.float32),
        scratch_types=[
            pltpu.VMEM((2, _D, _VB), jnp.float32),
            pltpu.VMEM((2, _VB // 2, 128), jnp.float32),
            pltpu.VMEM((_D, _TAIL), jnp.float32),
            pltpu.VMEM((_TAIL // 2, 128), jnp.float32),
            pltpu.SemaphoreType.DMA,
            pltpu.SemaphoreType.DMA,
        ],
        compiler_params=_CPARAMS,
    )
    def transpose_kernel(wt_hbm, w2, pin, pout, tin, tout, insem, outsem):
        cid = lax.axis_index("c")
        sid = lax.axis_index("s")
        wid = sid * _NC + cid

        nq = jnp.where(wid < _XTRA, _NFULL // _NW + 1, _NFULL // _NW)

        def fire_in(q):
            pltpu.async_copy(
                wt_hbm.at[:, pl.ds((q * _NW + wid) * _VB, _VB)],
                pin.at[q % 2], insem)

        def wait_in(q):
            pltpu.make_async_copy(
                wt_hbm.at[:, pl.ds(0, _VB)], pin.at[q % 2], insem).wait()

        def fire_out(q):
            pltpu.async_copy(
                pout.at[q % 2],
                w2.at[pl.ds((q * _NW + wid) * (_VB // 2), _VB // 2), :],
                outsem)

        def wait_out(q):
            pltpu.make_async_copy(
                pout.at[q % 2], w2.at[pl.ds(0, _VB // 2), :], outsem).wait()

        fire_in(0)

        def step(q, carry):
            wait_in(q)

            @pl.when(q + 1 < nq)
            def _():
                fire_in(q + 1)

            @pl.when(q >= 2)
            def _():
                wait_out(q)
            _pair_transp(pin.at[q % 2], pout.at[q % 2], _VB // 16)
            fire_out(q)
            return carry

        lax.fori_loop(0, nq, step, 0)
        wait_out(0)
        wait_out(1)

        @pl.when(wid == 4)
        def _():
            pltpu.sync_copy(wt_hbm.at[:, pl.ds(_NFULL * _VB, _TAIL)], tin)
            _pair_transp(tin, tout, _TAIL // 16)
            pltpu.sync_copy(
                tout, w2.at[pl.ds(_NFULL * (_VB // 2), _TAIL // 2), :])

    return transpose_kernel


@functools.cache
def _build_gather():
    mesh = plsc.VectorSubcoreMesh(core_axis_name="c", subcore_axis_name="s")

    @functools.partial(
        pl.kernel,
        mesh=mesh,
        out_type=jax.ShapeDtypeStruct((_H, _D, _B), jnp.float32),
        scratch_types=[
            pltpu.VMEM((4, 128), jnp.int32),
            pltpu.VMEM((4, 128), jnp.int32),
            pltpu.VMEM((4, 128, 128), jnp.float32),
            pltpu.VMEM((4, _D, 128), jnp.float32),
            pltpu.SemaphoreType.DMA,
            pltpu.SemaphoreType.DMA,
            pltpu.SemaphoreType.DMA,
            pltpu.SemaphoreType.DMA,
            pltpu.SemaphoreType.DMA,
            pltpu.SemaphoreType.DMA,
            pltpu.SemaphoreType.DMA,
            pltpu.SemaphoreType.DMA,
        ],
        compiler_params=_CPARAMS,
    )
    def gather_kernel(tok_hbm, w2, out_hbm, ix, ix2, rows, outb,
                      g0, g1, g2, g3, s0, s1, s2, s3):
        gsem = (g0, g1, g2, g3)
        ssem = (s0, s1, s2, s3)
        cid = lax.axis_index("c")
        sid = lax.axis_index("s")
        wid = sid * _NC + cid
        base = wid * _TASKS
        iota = lax.iota(jnp.int32, 16)

        def dest(t):
            g = base + t
            return g // 32, g % 32

        def fire_ix(t, k):
            j, ic = dest(t)
            pltpu.async_copy(tok_hbm.at[j, pl.ds(ic * 128, 128)],
                             ix.at[k], gsem[k])

        def wait_ix(k):
            pltpu.make_async_copy(tok_hbm.at[0, pl.ds(0, 128)], ix.at[k],
                                  gsem[k]).wait()

        def fire_g(k):
            for c in range(8):
                v = ix[k, pl.ds(c * 16, 16)]
                ix2[k, pl.ds(c * 16, 16)] = v // 2
            pltpu.async_copy(w2.at[ix2.at[k]], rows.at[k], gsem[k])

        def wait_g(k):
            pltpu.make_async_copy(w2.at[pl.ds(0, 128), :], rows.at[k],
                                  gsem[k]).wait()

        def extract(k2):
            offs = [(ix[k2, pl.ds(c * 16, 16)] % 2) * 64 for c in range(8)]
            lcs = [iota + 16 * c for c in range(8)]
            src = rows.at[k2]
            dst = outb.at[k2]

            @plsc.parallel_loop(0, 16, unroll=4)
            def diag(d0):
                dd = (iota + d0) % 16
                for dh in range(_D // 16):
                    dvec = dd + dh * 16
                    xs = [plsc.load_gather(src, [lcs[c], offs[c] + dvec])
                          for c in range(8)]
                    for c in range(8):
                        plsc.store_scatter(dst, [dvec, lcs[c]], xs[c])

        def fire_st(t, k):
            j, ic = dest(t)
            pltpu.async_copy(outb.at[k],
                             out_hbm.at[j, :, pl.ds(ic * 128, 128)],
                             ssem[k])

        def wait_st(k):
            pltpu.make_async_copy(outb.at[k],
                                  out_hbm.at[0, :, pl.ds(0, 128)],
                                  ssem[k]).wait()

        def position(t, p):
            k = p % 4
            k2 = (p + 2) % 4
            wait_ix(k)
            fire_g(k)
            if p >= 2:
                wait_g(k2)
                if p >= 6:
                    wait_st(k2)
                extract(k2)
                if t + 2 < _TASKS:
                    fire_ix(t + 2, k2)
                fire_st(t - 2, k2)
            else:
                fire_ix(t + 2, (p + 2) % 4)

        fire_ix(0, 0)
        fire_ix(1, 1)
        for p in range(8):
            position(p, p)

        def p2_step(i, carry):
            for k in range(4):
                t = i * 4 + k
                k2 = (k + 2) % 4
                wait_ix(k)
                fire_g(k)
                wait_g(k2)
                wait_st(k2)
                extract(k2)

                @pl.when(t + 2 < _TASKS)
                def _():
                    fire_ix(t + 2, k2)

                fire_st(t - 2, k2)
            return carry

        lax.fori_loop(2, _TASKS // 4, p2_step, 0)

        for e in range(2):
            k2 = (_TASKS - 2 + e) % 4
            wait_g(k2)
            wait_st(k2)
            extract(k2)
            fire_st(_TASKS - 2 + e, k2)
        for k in range(4):
            wait_st(k)

    return gather_kernel


@jax.jit
def _run(token_ids, weight):
    tokT = token_ids.astype(jnp.int32).T
    wT = weight.T
    w2 = _build_transpose()(wT)
    outT = _build_gather()(tokT, w2)
    return outT.transpose(2, 0, 1)


def kernel(token_ids, weight):
    return _run(token_ids, weight)


# speedup vs baseline: 1.0092x; 1.0092x over previous
import functools

import jax
import jax.numpy as jnp
from jax import lax
from jax.experimental import pallas as pl
from jax.experimental.pallas import tpu as pltpu
from jax.experimental.pallas import tpu_sc as plsc

_NC = 2
_NS = 16
_NW = _NC * _NS

_V = 1000000
_D = 64
_B = 4096
_H = 200

_VB = 128
_NFULL = _V // _VB
_TAIL = _V - _NFULL * _VB
_XTRA = _NFULL - (_NFULL // _NW) * _NW
_TASKS = (_H * (_B // 128)) // _NW

_CPARAMS = pltpu.CompilerParams(
    use_tc_tiling_on_sc=True, needs_layout_passes=False)


def _pair_transp(src, dst, chunks):
    iota = lax.iota(jnp.int32, 16)
    lcs = [iota + 16 * c for c in range(chunks)]
    prs = [(iota + 16 * c) // 2 for c in range(chunks)]
    colb = (iota % 2) * 64

    @plsc.parallel_loop(0, 16, unroll=2)
    def diag(d0):
        dd = (iota + d0) % 16
        for dh in range(_D // 16):
            dvec = dd + dh * 16
            colv = colb + dvec
            xs = [plsc.load_gather(src, [dvec, lcs[c]]) for c in range(chunks)]
            for c in range(chunks):
                plsc.store_scatter(dst, [prs[c], colv], xs[c])


@functools.cache
def _build_transpose():
    mesh = plsc.VectorSubcoreMesh(core_axis_name="c", subcore_axis_name="s")

    @functools.partial(
        pl.kernel,
        mesh=mesh,
        out_type=jax.ShapeDtypeStruct((_V // 2, 128), jnp.float32),
        scratch_types=[
            pltpu.VMEM((2, _D, _VB), jnp.float32),
            pltpu.VMEM((2, _VB // 2, 128), jnp.float32),
            pltpu.VMEM((_D, _TAIL), jnp.float32),
            pltpu.VMEM((_TAIL // 2, 128), jnp.float32),
            pltpu.SemaphoreType.DMA,
            pltpu.SemaphoreType.DMA,
        ],
        compiler_params=_CPARAMS,
    )
    def transpose_kernel(wt_hbm, w2, pin, pout, tin, tout, insem, outsem):
        cid = lax.axis_index("c")
        sid = lax.axis_index("s")
        wid = sid * _NC + cid

        nq = jnp.where(wid < _XTRA, _NFULL // _NW + 1, _NFULL // _NW)

        def fire_in(q):
            pltpu.async_copy(
                wt_hbm.at[:, pl.ds((q * _NW + wid) * _VB, _VB)],
                pin.at[q % 2], insem)

        def wait_in(q):
            pltpu.make_async_copy(
                wt_hbm.at[:, pl.ds(0, _VB)], pin.at[q % 2], insem).wait()

        def fire_out(q):
            pltpu.async_copy(
                pout.at[q % 2],
                w2.at[pl.ds((q * _NW + wid) * (_VB // 2), _VB // 2), :],
                outsem)

        def wait_out(q):
            pltpu.make_async_copy(
                pout.at[q % 2], w2.at[pl.ds(0, _VB // 2), :], outsem).wait()

        fire_in(0)

        def step(q, carry):
            wait_in(q)

            @pl.when(q + 1 < nq)
            def _():
                fire_in(q + 1)

            @pl.when(q >= 2)
            def _():
                wait_out(q)
            _pair_transp(pin.at[q % 2], pout.at[q % 2], _VB // 16)
            fire_out(q)
            return carry

        lax.fori_loop(0, nq, step, 0)
        wait_out(0)
        wait_out(1)

        @pl.when(wid == 4)
        def _():
            pltpu.sync_copy(wt_hbm.at[:, pl.ds(_NFULL * _VB, _TAIL)], tin)
            _pair_transp(tin, tout, _TAIL // 16)
            pltpu.sync_copy(
                tout, w2.at[pl.ds(_NFULL * (_VB // 2), _TAIL // 2), :])

    return transpose_kernel


@functools.cache
def _build_gather():
    mesh = plsc.VectorSubcoreMesh(core_axis_name="c", subcore_axis_name="s")

    @functools.partial(
        pl.kernel,
        mesh=mesh,
        out_type=jax.ShapeDtypeStruct((_H, _D, _B), jnp.float32),
        scratch_types=[
            pltpu.VMEM((4, 128), jnp.int32),
            pltpu.VMEM((4, 128), jnp.int32),
            pltpu.VMEM((4, 128, 128), jnp.float32),
            pltpu.VMEM((4, _D, 128), jnp.float32),
            pltpu.SemaphoreType.DMA,
            pltpu.SemaphoreType.DMA,
            pltpu.SemaphoreType.DMA,
            pltpu.SemaphoreType.DMA,
            pltpu.SemaphoreType.DMA,
            pltpu.SemaphoreType.DMA,
            pltpu.SemaphoreType.DMA,
            pltpu.SemaphoreType.DMA,
        ],
        compiler_params=_CPARAMS,
    )
    def gather_kernel(tok_hbm, w2, out_hbm, ix, ix2, rows, outb,
                      g0, g1, g2, g3, s0, s1, s2, s3):
        gsem = (g0, g1, g2, g3)
        ssem = (s0, s1, s2, s3)
        cid = lax.axis_index("c")
        sid = lax.axis_index("s")
        wid = sid * _NC + cid
        base = wid * _TASKS
        iota = lax.iota(jnp.int32, 16)

        def dest(t):
            g = base + t
            return g // 32, g % 32

        def fire_ix(t, k):
            j, ic = dest(t)
            pltpu.async_copy(tok_hbm.at[j, pl.ds(ic * 128, 128)],
                             ix.at[k], gsem[k])

        def wait_ix(k):
            pltpu.make_async_copy(tok_hbm.at[0, pl.ds(0, 128)], ix.at[k],
                                  gsem[k]).wait()

        def fire_g(k):
            for c in range(8):
                v = ix[k, pl.ds(c * 16, 16)]
                ix2[k, pl.ds(c * 16, 16)] = v // 2
            pltpu.async_copy(w2.at[ix2.at[k]], rows.at[k], gsem[k])

        def wait_g(k):
            pltpu.make_async_copy(w2.at[pl.ds(0, 128), :], rows.at[k],
                                  gsem[k]).wait()

        def extract(k2):
            offs = [(ix[k2, pl.ds(c * 16, 16)] % 2) * 64 for c in range(8)]
            lcs = [iota + 16 * c for c in range(8)]
            src = rows.at[k2]
            dst = outb.at[k2]

            @plsc.parallel_loop(0, 16, unroll=2)
            def diag(d0):
                dd = (iota + d0) % 16
                for dh in range(_D // 16):
                    dvec = dd + dh * 16
                    xs = [plsc.load_gather(src, [lcs[c], offs[c] + dvec])
                          for c in range(8)]
                    for c in range(8):
                        plsc.store_scatter(dst, [dvec, lcs[c]], xs[c])

        def fire_st(t, k):
            j, ic = dest(t)
            pltpu.async_copy(outb.at[k],
                             out_hbm.at[j, :, pl.ds(ic * 128, 128)],
                             ssem[k])

        def wait_st(k):
            pltpu.make_async_copy(outb.at[k],
                                  out_hbm.at[0, :, pl.ds(0, 128)],
                                  ssem[k]).wait()

        def position(t, p):
            k = p % 4
            k2 = (p + 2) % 4
            wait_ix(k)
            fire_g(k)
            if p >= 2:
                wait_g(k2)
                if p >= 6:
                    wait_st(k2)
                extract(k2)
                if t + 2 < _TASKS:
                    fire_ix(t + 2, k2)
                fire_st(t - 2, k2)
            else:
                fire_ix(t + 2, (p + 2) % 4)

        fire_ix(0, 0)
        fire_ix(1, 1)
        for p in range(8):
            position(p, p)

        def p2_step(i, carry):
            for k in range(4):
                t = i * 4 + k
                k2 = (k + 2) % 4
                wait_ix(k)
                fire_g(k)
                wait_g(k2)
                wait_st(k2)
                extract(k2)

                @pl.when(t + 2 < _TASKS)
                def _():
                    fire_ix(t + 2, k2)

                fire_st(t - 2, k2)
            return carry

        lax.fori_loop(2, _TASKS // 4, p2_step, 0)

        for e in range(2):
            k2 = (_TASKS - 2 + e) % 4
            wait_g(k2)
            wait_st(k2)
            extract(k2)
            fire_st(_TASKS - 2 + e, k2)
        for k in range(4):
            wait_st(k)

    return gather_kernel


@jax.jit
def _run(token_ids, weight):
    tokT = token_ids.astype(jnp.int32).T
    wT = weight.T
    w2 = _build_transpose()(wT)
    outT = _build_gather()(tokT, w2)
    return outT.transpose(2, 0, 1)


def kernel(token_ids, weight):
    return _run(token_ids, weight)


# speedup vs baseline: 1.1150x; 1.1048x over previous
import functools

import jax
import jax.numpy as jnp
from jax import lax
from jax.experimental import pallas as pl
from jax.experimental.pallas import tpu as pltpu
from jax.experimental.pallas import tpu_sc as plsc

_NC = 2
_NS = 16
_NW = _NC * _NS

_V = 1000000
_D = 64
_B = 4096
_H = 200

_VB = 256
_NFULL = _V // _VB
_TAIL = _V - _NFULL * _VB
_XTRA = _NFULL - (_NFULL // _NW) * _NW
_TASKS = (_H * (_B // 128)) // _NW

_CPARAMS = pltpu.CompilerParams(
    use_tc_tiling_on_sc=True, needs_layout_passes=False)


def _pair_transp(src, dst, chunks):
    iota = lax.iota(jnp.int32, 16)
    lcs = [iota + 16 * c for c in range(chunks)]
    prs = [(iota + 16 * c) // 2 for c in range(chunks)]
    colb = (iota % 2) * 64

    @plsc.parallel_loop(0, 16, unroll=2)
    def diag(d0):
        dd = (iota + d0) % 16
        for dh in range(_D // 16):
            dvec = dd + dh * 16
            colv = colb + dvec
            xs = [plsc.load_gather(src, [dvec, lcs[c]]) for c in range(chunks)]
            for c in range(chunks):
                plsc.store_scatter(dst, [prs[c], colv], xs[c])


@functools.cache
def _build_transpose():
    mesh = plsc.VectorSubcoreMesh(core_axis_name="c", subcore_axis_name="s")

    @functools.partial(
        pl.kernel,
        mesh=mesh,
        out_type=jax.ShapeDtypeStruct((_V // 2, 128), jnp.float32),
        scratch_types=[
            pltpu.VMEM((2, _D, _VB), jnp.float32),
            pltpu.VMEM((2, _VB // 2, 128), jnp.float32),
            pltpu.VMEM((_D, _TAIL), jnp.float32),
            pltpu.VMEM((_TAIL // 2, 128), jnp.float32),
            pltpu.SemaphoreType.DMA,
            pltpu.SemaphoreType.DMA,
        ],
        compiler_params=_CPARAMS,
    )
    def transpose_kernel(wt_hbm, w2, pin, pout, tin, tout, insem, outsem):
        cid = lax.axis_index("c")
        sid = lax.axis_index("s")
        wid = sid * _NC + cid

        nq = jnp.where(wid < _XTRA, _NFULL // _NW + 1, _NFULL // _NW)

        def fire_in(q):
            pltpu.async_copy(
                wt_hbm.at[:, pl.ds((q * _NW + wid) * _VB, _VB)],
                pin.at[q % 2], insem)

        def wait_in(q):
            pltpu.make_async_copy(
                wt_hbm.at[:, pl.ds(0, _VB)], pin.at[q % 2], insem).wait()

        def fire_out(q):
            pltpu.async_copy(
                pout.at[q % 2],
                w2.at[pl.ds((q * _NW + wid) * (_VB // 2), _VB // 2), :],
                outsem)

        def wait_out(q):
            pltpu.make_async_copy(
                pout.at[q % 2], w2.at[pl.ds(0, _VB // 2), :], outsem).wait()

        fire_in(0)

        def step(q, carry):
            wait_in(q)

            @pl.when(q + 1 < nq)
            def _():
                fire_in(q + 1)

            @pl.when(q >= 2)
            def _():
                wait_out(q)
            _pair_transp(pin.at[q % 2], pout.at[q % 2], _VB // 16)
            fire_out(q)
            return carry

        lax.fori_loop(0, nq, step, 0)
        wait_out(0)
        wait_out(1)

        @pl.when(wid == 4)
        def _():
            pltpu.sync_copy(wt_hbm.at[:, pl.ds(_NFULL * _VB, _TAIL)], tin)
            _pair_transp(tin, tout, _TAIL // 16)
            pltpu.sync_copy(
                tout, w2.at[pl.ds(_NFULL * (_VB // 2), _TAIL // 2), :])

    return transpose_kernel


@functools.cache
def _build_gather():
    mesh = plsc.VectorSubcoreMesh(core_axis_name="c", subcore_axis_name="s")

    @functools.partial(
        pl.kernel,
        mesh=mesh,
        out_type=jax.ShapeDtypeStruct((_H, _D, _B), jnp.float32),
        scratch_types=[
            pltpu.VMEM((4, 128), jnp.int32),
            pltpu.VMEM((4, 128), jnp.int32),
            pltpu.VMEM((4, 128, 128), jnp.float32),
            pltpu.VMEM((4, _D, 128), jnp.float32),
            pltpu.SemaphoreType.DMA,
            pltpu.SemaphoreType.DMA,
            pltpu.SemaphoreType.DMA,
            pltpu.SemaphoreType.DMA,
            pltpu.SemaphoreType.DMA,
            pltpu.SemaphoreType.DMA,
            pltpu.SemaphoreType.DMA,
            pltpu.SemaphoreType.DMA,
        ],
        compiler_params=_CPARAMS,
    )
    def gather_kernel(tok_hbm, w2, out_hbm, ix, ix2, rows, outb,
                      g0, g1, g2, g3, s0, s1, s2, s3):
        gsem = (g0, g1, g2, g3)
        ssem = (s0, s1, s2, s3)
        cid = lax.axis_index("c")
        sid = lax.axis_index("s")
        wid = sid * _NC + cid
        base = wid * _TASKS
        iota = lax.iota(jnp.int32, 16)

        def dest(t):
            g = base + t
            return g // 32, g % 32

        def fire_ix(t, k):
            j, ic = dest(t)
            pltpu.async_copy(tok_hbm.at[j, pl.ds(ic * 128, 128)],
                             ix.at[k], gsem[k])

        def wait_ix(k):
            pltpu.make_async_copy(tok_hbm.at[0, pl.ds(0, 128)], ix.at[k],
                                  gsem[k]).wait()

        def fire_g(k):
            for c in range(8):
                v = ix[k, pl.ds(c * 16, 16)]
                ix2[k, pl.ds(c * 16, 16)] = v // 2
            pltpu.async_copy(w2.at[ix2.at[k]], rows.at[k], gsem[k])

        def wait_g(k):
            pltpu.make_async_copy(w2.at[pl.ds(0, 128), :], rows.at[k],
                                  gsem[k]).wait()

        def extract(k2):
            offs = [(ix[k2, pl.ds(c * 16, 16)] % 2) * 64 for c in range(8)]
            lcs = [iota + 16 * c for c in range(8)]
            src = rows.at[k2]
            dst = outb.at[k2]

            @plsc.parallel_loop(0, 16, unroll=2)
            def diag(d0):
                dd = (iota + d0) % 16
                for dh in range(_D // 16):
                    dvec = dd + dh * 16
                    xs = [plsc.load_gather(src, [lcs[c], offs[c] + dvec])
                          for c in range(8)]
                    for c in range(8):
                        plsc.store_scatter(dst, [dvec, lcs[c]], xs[c])

        def fire_st(t, k):
            j, ic = dest(t)
            pltpu.async_copy(outb.at[k],
                             out_hbm.at[j, :, pl.ds(ic * 128, 128)],
                             ssem[k])

        def wait_st(k):
            pltpu.make_async_copy(outb.at[k],
                                  out_hbm.at[0, :, pl.ds(0, 128)],
                                  ssem[k]).wait()

        def position(t, p):
            k = p % 4
            k2 = (p + 2) % 4
            wait_ix(k)
            fire_g(k)
            if p >= 2:
                wait_g(k2)
                if p >= 6:
                    wait_st(k2)
                extract(k2)
                if t + 2 < _TASKS:
                    fire_ix(t + 2, k2)
                fire_st(t - 2, k2)
            else:
                fire_ix(t + 2, (p + 2) % 4)

        fire_ix(0, 0)
        fire_ix(1, 1)
        for p in range(8):
            position(p, p)

        def p2_step(i, carry):
            for k in range(4):
                t = i * 4 + k
                k2 = (k + 2) % 4
                wait_ix(k)
                fire_g(k)
                wait_g(k2)
                wait_st(k2)
                extract(k2)

                @pl.when(t + 2 < _TASKS)
                def _():
                    fire_ix(t + 2, k2)

                fire_st(t - 2, k2)
            return carry

        lax.fori_loop(2, _TASKS // 4, p2_step, 0)

        for e in range(2):
            k2 = (_TASKS - 2 + e) % 4
            wait_g(k2)
            wait_st(k2)
            extract(k2)
            fire_st(_TASKS - 2 + e, k2)
        for k in range(4):
            wait_st(k)

    return gather_kernel


@jax.jit
def _run(token_ids, weight):
    tokT = token_ids.astype(jnp.int32).T
    wT = weight.T
    w2 = _build_transpose()(wT)
    outT = _build_gather()(tokT, w2)
    return outT.transpose(2, 0, 1)


def kernel(token_ids, weight):
    return _run(token_ids, weight)


# speedup vs baseline: 1.1660x; 1.0458x over previous
import functools

import jax
import jax.numpy as jnp
from jax import lax
from jax.experimental import pallas as pl
from jax.experimental.pallas import tpu as pltpu
from jax.experimental.pallas import tpu_sc as plsc

_NC = 2
_NS = 16
_NW = _NC * _NS

_V = 1000000
_D = 64
_B = 4096
_H = 200

_VB = 256
_NFULL = _V // _VB
_TAIL = _V - _NFULL * _VB
_XTRA = _NFULL - (_NFULL // _NW) * _NW
_TASKS = (_H * (_B // 128)) // _NW

_CPARAMS = pltpu.CompilerParams(
    use_tc_tiling_on_sc=True, needs_layout_passes=False)


def _pair_transp(src, dst, chunks):
    iota = lax.iota(jnp.int32, 16)
    lcs = [iota + 16 * c for c in range(chunks)]
    prs = [(iota + 16 * c) // 2 for c in range(chunks)]
    colb = (iota % 2) * 64

    @plsc.parallel_loop(0, 16, unroll=2)
    def diag(d0):
        dd = (iota + d0) % 16
        for dh in range(_D // 16):
            dvec = dd + dh * 16
            colv = colb + dvec
            xs = [plsc.load_gather(src, [dvec, lcs[c]]) for c in range(chunks)]
            for c in range(chunks):
                plsc.store_scatter(dst, [prs[c], colv], xs[c])


@functools.cache
def _build_transpose():
    mesh = plsc.VectorSubcoreMesh(core_axis_name="c", subcore_axis_name="s")

    @functools.partial(
        pl.kernel,
        mesh=mesh,
        out_type=jax.ShapeDtypeStruct((_V // 2, 128), jnp.float32),
        scratch_types=[
            pltpu.VMEM((3, _D, _VB), jnp.float32),
            pltpu.VMEM((3, _VB // 2, 128), jnp.float32),
            pltpu.VMEM((_D, _TAIL), jnp.float32),
            pltpu.VMEM((_TAIL // 2, 128), jnp.float32),
            pltpu.SemaphoreType.DMA,
            pltpu.SemaphoreType.DMA,
        ],
        compiler_params=_CPARAMS,
    )
    def transpose_kernel(wt_hbm, w2, pin, pout, tin, tout, insem, outsem):
        cid = lax.axis_index("c")
        sid = lax.axis_index("s")
        wid = sid * _NC + cid

        nq = jnp.where(wid < _XTRA, _NFULL // _NW + 1, _NFULL // _NW)

        def fire_in(q):
            pltpu.async_copy(
                wt_hbm.at[:, pl.ds((q * _NW + wid) * _VB, _VB)],
                pin.at[q % 3], insem)

        def wait_in(q):
            pltpu.make_async_copy(
                wt_hbm.at[:, pl.ds(0, _VB)], pin.at[q % 3], insem).wait()

        def fire_out(q):
            pltpu.async_copy(
                pout.at[q % 3],
                w2.at[pl.ds((q * _NW + wid) * (_VB // 2), _VB // 2), :],
                outsem)

        def wait_out(q):
            pltpu.make_async_copy(
                pout.at[q % 3], w2.at[pl.ds(0, _VB // 2), :], outsem).wait()

        fire_in(0)
        fire_in(1)

        def step(q, carry):
            wait_in(q)

            @pl.when(q + 2 < nq)
            def _():
                fire_in(q + 2)

            @pl.when(q >= 3)
            def _():
                wait_out(q)
            _pair_transp(pin.at[q % 3], pout.at[q % 3], _VB // 16)
            fire_out(q)
            return carry

        lax.fori_loop(0, nq, step, 0)
        wait_out(0)
        wait_out(1)
        wait_out(2)

        @pl.when(wid == 4)
        def _():
            pltpu.sync_copy(wt_hbm.at[:, pl.ds(_NFULL * _VB, _TAIL)], tin)
            _pair_transp(tin, tout, _TAIL // 16)
            pltpu.sync_copy(
                tout, w2.at[pl.ds(_NFULL * (_VB // 2), _TAIL // 2), :])

    return transpose_kernel


@functools.cache
def _build_gather():
    mesh = plsc.VectorSubcoreMesh(core_axis_name="c", subcore_axis_name="s")

    @functools.partial(
        pl.kernel,
        mesh=mesh,
        out_type=jax.ShapeDtypeStruct((_H, _D, _B), jnp.float32),
        scratch_types=[
            pltpu.VMEM((4, 128), jnp.int32),
            pltpu.VMEM((4, 128), jnp.int32),
            pltpu.VMEM((4, 128, 128), jnp.float32),
            pltpu.VMEM((4, _D, 128), jnp.float32),
            pltpu.SemaphoreType.DMA,
            pltpu.SemaphoreType.DMA,
            pltpu.SemaphoreType.DMA,
            pltpu.SemaphoreType.DMA,
            pltpu.SemaphoreType.DMA,
            pltpu.SemaphoreType.DMA,
            pltpu.SemaphoreType.DMA,
            pltpu.SemaphoreType.DMA,
        ],
        compiler_params=_CPARAMS,
    )
    def gather_kernel(tok_hbm, w2, out_hbm, ix, ix2, rows, outb,
                      g0, g1, g2, g3, s0, s1, s2, s3):
        gsem = (g0, g1, g2, g3)
        ssem = (s0, s1, s2, s3)
        cid = lax.axis_index("c")
        sid = lax.axis_index("s")
        wid = sid * _NC + cid
        base = wid * _TASKS
        iota = lax.iota(jnp.int32, 16)

        def dest(t):
            g = base + t
            return g // 32, g % 32

        def fire_ix(t, k):
            j, ic = dest(t)
            pltpu.async_copy(tok_hbm.at[j, pl.ds(ic * 128, 128)],
                             ix.at[k], gsem[k])

        def wait_ix(k):
            pltpu.make_async_copy(tok_hbm.at[0, pl.ds(0, 128)], ix.at[k],
                                  gsem[k]).wait()

        def fire_g(k):
            for c in range(8):
                v = ix[k, pl.ds(c * 16, 16)]
                ix2[k, pl.ds(c * 16, 16)] = v // 2
            pltpu.async_copy(w2.at[ix2.at[k]], rows.at[k], gsem[k])

        def wait_g(k):
            pltpu.make_async_copy(w2.at[pl.ds(0, 128), :], rows.at[k],
                                  gsem[k]).wait()

        def extract(k2):
            offs = [(ix[k2, pl.ds(c * 16, 16)] % 2) * 64 for c in range(8)]
            lcs = [iota + 16 * c for c in range(8)]
            src = rows.at[k2]
            dst = outb.at[k2]

            @plsc.parallel_loop(0, 16, unroll=2)
            def diag(d0):
                dd = (iota + d0) % 16
                for dh in range(_D // 16):
                    dvec = dd + dh * 16
                    xs = [plsc.load_gather(src, [lcs[c], offs[c] + dvec])
                          for c in range(8)]
                    for c in range(8):
                        plsc.store_scatter(dst, [dvec, lcs[c]], xs[c])

        def fire_st(t, k):
            j, ic = dest(t)
            pltpu.async_copy(outb.at[k],
                             out_hbm.at[j, :, pl.ds(ic * 128, 128)],
                             ssem[k])

        def wait_st(k):
            pltpu.make_async_copy(outb.at[k],
                                  out_hbm.at[0, :, pl.ds(0, 128)],
                                  ssem[k]).wait()

        def position(t, p):
            k = p % 4
            k2 = (p + 2) % 4
            wait_ix(k)
            fire_g(k)
            if p >= 2:
                wait_g(k2)
                if p >= 6:
                    wait_st(k2)
                extract(k2)
                if t + 2 < _TASKS:
                    fire_ix(t + 2, k2)
                fire_st(t - 2, k2)
            else:
                fire_ix(t + 2, (p + 2) % 4)

        fire_ix(0, 0)
        fire_ix(1, 1)
        for p in range(8):
            position(p, p)

        def p2_step(i, carry):
            for k in range(4):
                t = i * 4 + k
                k2 = (k + 2) % 4
                wait_ix(k)
                fire_g(k)
                wait_g(k2)
                wait_st(k2)
                extract(k2)

                @pl.when(t + 2 < _TASKS)
                def _():
                    fire_ix(t + 2, k2)

                fire_st(t - 2, k2)
            return carry

        lax.fori_loop(2, _TASKS // 4, p2_step, 0)

        for e in range(2):
            k2 = (_TASKS - 2 + e) % 4
            wait_g(k2)
            wait_st(k2)
            extract(k2)
            fire_st(_TASKS - 2 + e, k2)
        for k in range(4):
            wait_st(k)

    return gather_kernel


@jax.jit
def _run(token_ids, weight):
    tokT = token_ids.astype(jnp.int32).T
    wT = weight.T
    w2 = _build_transpose()(wT)
    outT = _build_gather()(tokT, w2)
    return outT.transpose(2, 0, 1)


def kernel(token_ids, weight):
    return _run(token_ids, weight)
